# baseline (speedup 1.0000x reference)
import jax
import jax.numpy as jnp
from jax import lax
from jax.experimental import pallas as pl
from jax.experimental.pallas import tpu as pltpu
from jax.experimental.pallas import tpu_sc as plsc

LANES = 16
BLK = 100
JPC = 2
NC = 2
NS = 16
NW = NC * NS


def _body(x_hbm, tok_hbm, pos_hbm, out_hbm, idx_v, rows_v, pos_v, gsem):
    emb = tok_hbm.shape[1]
    wid = lax.axis_index("s") * NC + lax.axis_index("c")
    nblocks = x_hbm.shape[0]
    blocks_w = nblocks // NW
    nchunks = blocks_w // JPC

    pltpu.sync_copy(pos_hbm, pos_v)

    def chunk_body(g, carry):
        row = wid * blocks_w + g * JPC
        pltpu.sync_copy(x_hbm.at[pl.ds(row, JPC)], idx_v)
        cps = [
            pltpu.async_copy(tok_hbm.at[idx_v.at[j]], rows_v.at[j], gsem)
            for j in range(JPC)
        ]
        for cp in cps:
            cp.wait()

        def add_body(r, c):
            for j in range(JPC):
                for k in range(emb // LANES):
                    sl = pl.ds(k * LANES, LANES)
                    rows_v[j, r, sl] = rows_v[j, r, sl] + pos_v[j, r, sl]
            return c

        lax.fori_loop(0, BLK, add_body, 0, unroll=2)

        pltpu.sync_copy(rows_v, out_hbm.at[pl.ds(row, JPC)])
        return carry

    lax.fori_loop(0, nchunks, chunk_body, 0)


def kernel(x, tok_table, pos_table):
    b, s = x.shape
    v, e = tok_table.shape
    n = b * s
    assert s == JPC * BLK and n % (NW * JPC * BLK) == 0 and e % LANES == 0

    x2 = x.reshape(n // BLK, BLK)
    pos2 = pos_table[:s].reshape(JPC, BLK, e)

    out = pl.kernel(
        _body,
        out_type=jax.ShapeDtypeStruct((n // BLK, BLK, e), jnp.float32),
        mesh=plsc.VectorSubcoreMesh(core_axis_name="c", subcore_axis_name="s"),
        compiler_params=pltpu.CompilerParams(use_tc_tiling_on_sc=False),
        scratch_types=[
            pltpu.VMEM((JPC, BLK), jnp.int32),
            pltpu.VMEM((JPC, BLK, e), jnp.float32),
            pltpu.VMEM((JPC, BLK, e), jnp.float32),
            pltpu.SemaphoreType.DMA,
        ],
    )(x2, tok_table, pos2)
    return out.reshape(b, s, e)


# speedup vs baseline: 1.1757x; 1.1757x over previous
import jax
import jax.numpy as jnp
from jax import lax
from jax.experimental import pallas as pl
from jax.experimental.pallas import tpu as pltpu
from jax.experimental.pallas import tpu_sc as plsc

LANES = 16
BLK = 100
JPC = 2
NC = 2
NS = 16
NW = NC * NS


def _body(x_hbm, tok_hbm, pos_hbm, out_hbm,
          idx_all, rows_v, pos_v, gsem0, gsem1, osem0, osem1):
    emb = tok_hbm.shape[1]
    wid = lax.axis_index("s") * NC + lax.axis_index("c")
    nblocks = x_hbm.shape[0]
    blocks_w = nblocks // NW
    nchunks = blocks_w // JPC
    gsems = (gsem0, gsem1)
    osems = (osem0, osem1)

    pltpu.sync_copy(x_hbm.at[pl.ds(wid * blocks_w, blocks_w)], idx_all)
    pltpu.sync_copy(pos_hbm, pos_v)

    def start_gather(c, b):
        for j in range(JPC):
            pltpu.async_copy(
                tok_hbm.at[idx_all.at[c * JPC + j]], rows_v.at[b, j], gsems[b])

    start_gather(0, 0)

    def outer(g, carry):
        for b in range(2):
            c = g * 2 + b

            @pl.when(c >= 1)
            def _():
                pltpu.make_async_copy(
                    rows_v.at[1 - b], out_hbm.at[pl.ds(0, JPC)],
                    osems[1 - b]).wait()

            @pl.when(c + 1 < nchunks)
            def _():
                start_gather(c + 1, 1 - b)

            pltpu.make_async_copy(
                out_hbm.at[pl.ds(0, JPC)], rows_v.at[b], gsems[b]).wait()

            def add_body(r, cr):
                for j in range(JPC):
                    for k in range(emb // LANES):
                        sl = pl.ds(k * LANES, LANES)
                        plsc.addupdate(rows_v.at[b, j, r, sl], pos_v[j, r, sl])
                return cr

            lax.fori_loop(0, BLK, add_body, 0, unroll=4)

            row = wid * blocks_w + c * JPC
            pltpu.async_copy(rows_v.at[b], out_hbm.at[pl.ds(row, JPC)],
                             osems[b])
        return carry

    lax.fori_loop(0, nchunks // 2, outer, 0)

    pltpu.make_async_copy(
        rows_v.at[1], out_hbm.at[pl.ds(0, JPC)], osems[1]).wait()


def kernel(x, tok_table, pos_table):
    b, s = x.shape
    v, e = tok_table.shape
    n = b * s
    assert s == JPC * BLK and e % LANES == 0
    assert n % (NW * 2 * JPC * BLK) == 0

    x2 = x.reshape(n // BLK, BLK)
    pos2 = pos_table[:s].reshape(JPC, BLK, e)

    out = pl.kernel(
        _body,
        out_type=jax.ShapeDtypeStruct((n // BLK, BLK, e), jnp.float32),
        mesh=plsc.VectorSubcoreMesh(core_axis_name="c", subcore_axis_name="s"),
        compiler_params=pltpu.CompilerParams(use_tc_tiling_on_sc=False),
        scratch_types=[
            pltpu.VMEM((n // BLK // NW, BLK), jnp.int32),
            pltpu.VMEM((2, JPC, BLK, e), jnp.float32),
            pltpu.VMEM((JPC, BLK, e), jnp.float32),
            pltpu.SemaphoreType.DMA,
            pltpu.SemaphoreType.DMA,
            pltpu.SemaphoreType.DMA,
            pltpu.SemaphoreType.DMA,
        ],
    )(x2, tok_table, pos2)
    return out.reshape(b, s, e)
